# Initial kernel scaffold; baseline (speedup 1.0000x reference)
import jax
import jax.numpy as jnp
from jax.experimental import pallas as pl


def kernel(x, edge_index, W, b):
    raise NotImplementedError("write your pallas kernel here")


# speedup vs baseline: 7.9200x; 7.9200x over previous
import functools

import jax
import jax.numpy as jnp
from jax import lax
from jax.experimental import pallas as pl
from jax.experimental.pallas import tpu as pltpu
from jax.experimental.pallas import tpu_sc as plsc

D = 128
CHUNK = 128
NCORES = 2
NSUB = 16
NW = NCORES * NSUB


def _make_deg(n2, cpw):
    rpt = n2 // NSUB
    nzc = rpt // CHUNK
    mesh = plsc.VectorSubcoreMesh(core_axis_name="c", subcore_axis_name="s")

    @functools.partial(
        pl.kernel,
        out_type=jax.ShapeDtypeStruct((NCORES, n2, 16), jnp.float32),
        mesh=mesh,
        scratch_types=[
            pltpu.VMEM((cpw, CHUNK), jnp.int32),
            pltpu.VMEM((CHUNK, 16), jnp.float32),
            pltpu.VMEM((CHUNK, 16), jnp.float32),
            pltpu.VMEM_SHARED((n2, 16), jnp.float32),
        ],
    )
    def deg(dsts, out, didx, ones_rows, zbuf, dacc):
        cid = lax.axis_index("c")
        sid = lax.axis_index("s")
        wid = sid * NCORES + cid
        lanes = lax.iota(jnp.int32, 16)
        onehot = jnp.where(lanes == 0, 1.0, 0.0).astype(jnp.float32)
        zero16 = jnp.zeros((16,), jnp.float32)

        def fill(r, carry):
            ones_rows[r, :] = onehot
            zbuf[r, :] = zero16
            return carry

        lax.fori_loop(0, CHUNK, fill, 0)
        for k in range(nzc):
            pltpu.sync_copy(zbuf, dacc.at[pl.ds(sid * rpt + k * CHUNK, CHUNK)])
        plsc.subcore_barrier()
        pltpu.sync_copy(dsts.at[pl.ds(wid * cpw, cpw)], didx)

        def body(c, carry):
            pltpu.sync_copy(ones_rows, dacc.at[didx.at[c]], add=True)
            return carry

        lax.fori_loop(0, cpw, body, 0)
        plsc.subcore_barrier()
        for k in range(nzc):
            base = sid * rpt + k * CHUNK
            pltpu.sync_copy(dacc.at[pl.ds(base, CHUNK)], zbuf)
            pltpu.sync_copy(zbuf, out.at[cid, pl.ds(base, CHUNK)])

    return deg


def _make_hop(n2, cpw):
    rpt = n2 // NSUB
    nzc = rpt // CHUNK
    mesh = plsc.VectorSubcoreMesh(core_axis_name="c", subcore_axis_name="s")

    @functools.partial(
        pl.kernel,
        out_type=jax.ShapeDtypeStruct((NCORES, n2, D), jnp.float32),
        mesh=mesh,
        scratch_types=[
            pltpu.VMEM((cpw, CHUNK), jnp.int32),
            pltpu.VMEM((cpw, CHUNK), jnp.int32),
            pltpu.VMEM((CHUNK, D), jnp.float32),
            pltpu.VMEM_SHARED((n2, D), jnp.float32),
            pltpu.SemaphoreType.DMA,
        ],
    )
    def hop(table, srcs, dsts, out, sidx, didx, rows, acc, sem):
        cid = lax.axis_index("c")
        sid = lax.axis_index("s")
        wid = sid * NCORES + cid
        zero16 = jnp.zeros((16,), jnp.float32)

        def zrow(r, carry):
            for j in range(D // 16):
                rows[r, pl.ds(j * 16, 16)] = zero16
            return carry

        lax.fori_loop(0, CHUNK, zrow, 0)
        for k in range(nzc):
            pltpu.sync_copy(rows, acc.at[pl.ds(sid * rpt + k * CHUNK, CHUNK)])
        plsc.subcore_barrier()
        pltpu.sync_copy(srcs.at[pl.ds(wid * cpw, cpw)], sidx)
        pltpu.sync_copy(dsts.at[pl.ds(wid * cpw, cpw)], didx)

        def body(c, carry):
            pltpu.async_copy(table.at[sidx.at[c]], rows, sem).wait()
            pltpu.sync_copy(rows, acc.at[didx.at[c]], add=True)
            return carry

        lax.fori_loop(0, cpw, body, 0)
        plsc.subcore_barrier()
        for k in range(nzc):
            base = sid * rpt + k * CHUNK
            pltpu.sync_copy(acc.at[pl.ds(base, CHUNK)], rows)
            pltpu.sync_copy(rows, out.at[cid, pl.ds(base, CHUNK)])

    return hop


def _prep(xp, d0, d1):
    n2 = xp.shape[0]
    blk = 1024
    def body(x_ref, d0_ref, d1_ref, x1_ref, dis_ref, dinv_ref):
        deg = d0_ref[...] + d1_ref[...] + 1.0
        dis = lax.rsqrt(deg)
        x1_ref[...] = x_ref[...] * dis
        dis_ref[...] = dis
        dinv_ref[...] = 1.0 / deg
    return pl.pallas_call(
        body,
        grid=(n2 // blk,),
        in_specs=[pl.BlockSpec((blk, D), lambda i: (i, 0)),
                  pl.BlockSpec((blk, 1), lambda i: (i, 0)),
                  pl.BlockSpec((blk, 1), lambda i: (i, 0))],
        out_specs=[pl.BlockSpec((blk, D), lambda i: (i, 0)),
                   pl.BlockSpec((blk, 1), lambda i: (i, 0)),
                   pl.BlockSpec((blk, 1), lambda i: (i, 0))],
        out_shape=[jax.ShapeDtypeStruct((n2, D), jnp.float32),
                   jax.ShapeDtypeStruct((n2, 1), jnp.float32),
                   jax.ShapeDtypeStruct((n2, 1), jnp.float32)],
    )(xp, d0, d1)


def _mid(p0, p1, x1, dinv):
    n2 = x1.shape[0]
    blk = 1024
    def body(a_ref, b_ref, x1_ref, dinv_ref, o_ref):
        o_ref[...] = (a_ref[...] + b_ref[...] + x1_ref[...]) * dinv_ref[...]
    return pl.pallas_call(
        body,
        grid=(n2 // blk,),
        in_specs=[pl.BlockSpec((blk, D), lambda i: (i, 0)),
                  pl.BlockSpec((blk, D), lambda i: (i, 0)),
                  pl.BlockSpec((blk, D), lambda i: (i, 0)),
                  pl.BlockSpec((blk, 1), lambda i: (i, 0))],
        out_specs=pl.BlockSpec((blk, D), lambda i: (i, 0)),
        out_shape=jax.ShapeDtypeStruct((n2, D), jnp.float32),
    )(p0, p1, x1, dinv)


def _final(q0, q1, x3, dis, W, b, n):
    blk = 1000
    def body(a_ref, b2_ref, x3_ref, dis_ref, w_ref, bias_ref, o_ref):
        h = (a_ref[...] + b2_ref[...] + x3_ref[...]) * dis_ref[...]
        z = lax.dot_general(h, w_ref[...], (((1,), (1,)), ((), ())),
                            preferred_element_type=jnp.float32)
        z = z + bias_ref[...]
        m = jnp.max(z, axis=1, keepdims=True)
        e = jnp.exp(z - m)
        s = jnp.sum(e, axis=1, keepdims=True)
        o_ref[...] = z - m - jnp.log(s)
    return pl.pallas_call(
        body,
        grid=(n // blk,),
        in_specs=[pl.BlockSpec((blk, D), lambda i: (i, 0)),
                  pl.BlockSpec((blk, D), lambda i: (i, 0)),
                  pl.BlockSpec((blk, D), lambda i: (i, 0)),
                  pl.BlockSpec((blk, 1), lambda i: (i, 0)),
                  pl.BlockSpec((D, D), lambda i: (0, 0)),
                  pl.BlockSpec((1, D), lambda i: (0, 0))],
        out_specs=pl.BlockSpec((blk, D), lambda i: (i, 0)),
        out_shape=jax.ShapeDtypeStruct((n, D), jnp.float32),
    )(q0, q1, x3, dis, W, b)


def kernel(x, edge_index, W, b):
    n, d = x.shape
    e = edge_index.shape[1]
    n2 = ((n + 1 + 2047) // 2048) * 2048
    cpw = -(-e // (NW * CHUNK))
    if cpw % 2:
        cpw += 1
    ep = NW * CHUNK * cpw
    src = edge_index[0].astype(jnp.int32)
    dst = edge_index[1].astype(jnp.int32)
    padn = ep - e
    src2 = jnp.concatenate(
        [src, jnp.zeros((padn,), jnp.int32)]).reshape(NW * cpw, CHUNK)
    dst2 = jnp.concatenate(
        [dst, jnp.full((padn,), n, jnp.int32)]).reshape(NW * cpw, CHUNK)
    xp = jnp.pad(x, ((0, n2 - n), (0, 0)))

    degp = _make_deg(n2, cpw)(dst2)
    d0 = degp[0, :, 0:1]
    d1 = degp[1, :, 0:1]
    x1, dis, dinv = _prep(xp, d0, d1)

    hop = _make_hop(n2, cpw)
    p = hop(x1, src2, dst2)
    x3 = _mid(p[0], p[1], x1, dinv)
    q = hop(x3, src2, dst2)
    out = _final(q[0][:n], q[1][:n], x3[:n], dis[:n], W,
                 b.reshape(1, D), n)
    return out


# speedup vs baseline: 9.1899x; 1.1603x over previous
import functools

import jax
import jax.numpy as jnp
from jax import lax
from jax.experimental import pallas as pl
from jax.experimental.pallas import tpu as pltpu
from jax.experimental.pallas import tpu_sc as plsc

D = 128
CHUNK = 128
NCORES = 2
NSUB = 16
NW = NCORES * NSUB


def _make_deg(n2, cpw):
    rpt = n2 // NSUB
    nzc = rpt // CHUNK
    mesh = plsc.VectorSubcoreMesh(core_axis_name="c", subcore_axis_name="s")

    @functools.partial(
        pl.kernel,
        out_type=jax.ShapeDtypeStruct((NCORES, n2, 16), jnp.float32),
        mesh=mesh,
        scratch_types=[
            pltpu.VMEM((cpw, CHUNK), jnp.int32),
            pltpu.VMEM((CHUNK, 16), jnp.float32),
            pltpu.VMEM((CHUNK, 16), jnp.float32),
            pltpu.VMEM_SHARED((n2, 16), jnp.float32),
        ],
    )
    def deg(dsts, out, didx, ones_rows, zbuf, dacc):
        cid = lax.axis_index("c")
        sid = lax.axis_index("s")
        wid = sid * NCORES + cid
        lanes = lax.iota(jnp.int32, 16)
        onehot = jnp.where(lanes == 0, 1.0, 0.0).astype(jnp.float32)
        zero16 = jnp.zeros((16,), jnp.float32)

        def fill(r, carry):
            ones_rows[r, :] = onehot
            zbuf[r, :] = zero16
            return carry

        lax.fori_loop(0, CHUNK, fill, 0)
        for k in range(nzc):
            pltpu.sync_copy(zbuf, dacc.at[pl.ds(sid * rpt + k * CHUNK, CHUNK)])
        plsc.subcore_barrier()
        pltpu.sync_copy(dsts.at[pl.ds(wid * cpw, cpw)], didx)

        def body(c, carry):
            pltpu.sync_copy(ones_rows, dacc.at[didx.at[c]], add=True)
            return carry

        lax.fori_loop(0, cpw, body, 0)
        plsc.subcore_barrier()
        for k in range(nzc):
            base = sid * rpt + k * CHUNK
            pltpu.sync_copy(dacc.at[pl.ds(base, CHUNK)], zbuf)
            pltpu.sync_copy(zbuf, out.at[cid, pl.ds(base, CHUNK)])

    return deg


NBUF = 2


def _make_hop(n2, cpw):
    rpt = n2 // NSUB
    nzc = rpt // CHUNK
    ngrp = cpw // NBUF
    mesh = plsc.VectorSubcoreMesh(core_axis_name="c", subcore_axis_name="s")

    @functools.partial(
        pl.kernel,
        out_type=jax.ShapeDtypeStruct((NCORES, n2, D), jnp.float32),
        mesh=mesh,
        scratch_types=[
            pltpu.VMEM((cpw, CHUNK), jnp.int32),
            pltpu.VMEM((NBUF, CHUNK), jnp.int32),
            pltpu.VMEM((NBUF, CHUNK), jnp.int32),
            pltpu.VMEM((CHUNK, D), jnp.float32),
            pltpu.VMEM((CHUNK, D), jnp.float32),
            pltpu.VMEM_SHARED((n2, D), jnp.float32),
            pltpu.SemaphoreType.DMA,
            pltpu.SemaphoreType.DMA,
        ],
    )
    def hop(table, packed, out, pidx, sbuf, dbuf, r0, r1, acc, s0, s1):
        bufs = (r0, r1)
        sems = (s0, s1)
        cid = lax.axis_index("c")
        sid = lax.axis_index("s")
        wid = sid * NCORES + cid
        zero16 = jnp.zeros((16,), jnp.float32)

        def zrow(r, carry):
            for j in range(D // 16):
                r0[r, pl.ds(j * 16, 16)] = zero16
            return carry

        lax.fori_loop(0, CHUNK, zrow, 0)
        for k in range(nzc):
            pltpu.sync_copy(r0, acc.at[pl.ds(sid * rpt + k * CHUNK, CHUNK)])
        plsc.subcore_barrier()
        pltpu.sync_copy(packed.at[pl.ds(wid * cpw, cpw)], pidx)

        def unpack(c, b):
            for j in range(CHUNK // 16):
                v = pidx[c, pl.ds(j * 16, 16)]
                sbuf[b, pl.ds(j * 16, 16)] = jnp.bitwise_and(v, 0xFFFF)
                dbuf[b, pl.ds(j * 16, 16)] = jnp.right_shift(v, 16)

        for b in range(NBUF):
            unpack(b, b)
            pltpu.async_copy(table.at[sbuf.at[b]], bufs[b], sems[b])

        def group(g, carry):
            for b in range(NBUF):
                c = g * NBUF + b
                pltpu.make_async_copy(
                    table.at[sbuf.at[b]], bufs[b], sems[b]).wait()
                pltpu.sync_copy(bufs[b], acc.at[dbuf.at[b]], add=True)
                unpack(c + NBUF, b)
                pltpu.async_copy(table.at[sbuf.at[b]], bufs[b], sems[b])
            return carry

        lax.fori_loop(0, ngrp - 1, group, 0)
        for b in range(NBUF):
            pltpu.make_async_copy(
                table.at[sbuf.at[b]], bufs[b], sems[b]).wait()
            pltpu.sync_copy(bufs[b], acc.at[dbuf.at[b]], add=True)
        plsc.subcore_barrier()
        for k in range(nzc):
            base = sid * rpt + k * CHUNK
            pltpu.sync_copy(acc.at[pl.ds(base, CHUNK)], r0)
            pltpu.sync_copy(r0, out.at[cid, pl.ds(base, CHUNK)])

    return hop


def _prep(xp, d0, d1):
    n2 = xp.shape[0]
    blk = 1024
    def body(x_ref, d0_ref, d1_ref, x1_ref, dis_ref, dinv_ref):
        deg = d0_ref[...] + d1_ref[...] + 1.0
        dis = lax.rsqrt(deg)
        x1_ref[...] = x_ref[...] * dis
        dis_ref[...] = dis
        dinv_ref[...] = 1.0 / deg
    return pl.pallas_call(
        body,
        grid=(n2 // blk,),
        in_specs=[pl.BlockSpec((blk, D), lambda i: (i, 0)),
                  pl.BlockSpec((blk, 1), lambda i: (i, 0)),
                  pl.BlockSpec((blk, 1), lambda i: (i, 0))],
        out_specs=[pl.BlockSpec((blk, D), lambda i: (i, 0)),
                   pl.BlockSpec((blk, 1), lambda i: (i, 0)),
                   pl.BlockSpec((blk, 1), lambda i: (i, 0))],
        out_shape=[jax.ShapeDtypeStruct((n2, D), jnp.float32),
                   jax.ShapeDtypeStruct((n2, 1), jnp.float32),
                   jax.ShapeDtypeStruct((n2, 1), jnp.float32)],
    )(xp, d0, d1)


def _mid(p0, p1, x1, dinv):
    n2 = x1.shape[0]
    blk = 1024
    def body(a_ref, b_ref, x1_ref, dinv_ref, o_ref):
        o_ref[...] = (a_ref[...] + b_ref[...] + x1_ref[...]) * dinv_ref[...]
    return pl.pallas_call(
        body,
        grid=(n2 // blk,),
        in_specs=[pl.BlockSpec((blk, D), lambda i: (i, 0)),
                  pl.BlockSpec((blk, D), lambda i: (i, 0)),
                  pl.BlockSpec((blk, D), lambda i: (i, 0)),
                  pl.BlockSpec((blk, 1), lambda i: (i, 0))],
        out_specs=pl.BlockSpec((blk, D), lambda i: (i, 0)),
        out_shape=jax.ShapeDtypeStruct((n2, D), jnp.float32),
    )(p0, p1, x1, dinv)


def _final(q0, q1, x3, dis, W, b, n):
    blk = 1000
    def body(a_ref, b2_ref, x3_ref, dis_ref, w_ref, bias_ref, o_ref):
        h = (a_ref[...] + b2_ref[...] + x3_ref[...]) * dis_ref[...]
        z = lax.dot_general(h, w_ref[...], (((1,), (1,)), ((), ())),
                            preferred_element_type=jnp.float32)
        z = z + bias_ref[...]
        m = jnp.max(z, axis=1, keepdims=True)
        e = jnp.exp(z - m)
        s = jnp.sum(e, axis=1, keepdims=True)
        o_ref[...] = z - m - jnp.log(s)
    return pl.pallas_call(
        body,
        grid=(n // blk,),
        in_specs=[pl.BlockSpec((blk, D), lambda i: (i, 0)),
                  pl.BlockSpec((blk, D), lambda i: (i, 0)),
                  pl.BlockSpec((blk, D), lambda i: (i, 0)),
                  pl.BlockSpec((blk, 1), lambda i: (i, 0)),
                  pl.BlockSpec((D, D), lambda i: (0, 0)),
                  pl.BlockSpec((1, D), lambda i: (0, 0))],
        out_specs=pl.BlockSpec((blk, D), lambda i: (i, 0)),
        out_shape=jax.ShapeDtypeStruct((n, D), jnp.float32),
    )(q0, q1, x3, dis, W, b)


def kernel(x, edge_index, W, b):
    n, d = x.shape
    e = edge_index.shape[1]
    n2 = ((n + 1 + 2047) // 2048) * 2048
    cpw = -(-e // (NW * CHUNK))
    cpw = -(-cpw // NBUF) * NBUF
    ep = NW * CHUNK * cpw
    src = edge_index[0].astype(jnp.int32)
    dst = edge_index[1].astype(jnp.int32)
    padn = ep - e
    src2 = jnp.concatenate(
        [src, jnp.zeros((padn,), jnp.int32)]).reshape(NW * cpw, CHUNK)
    dst2 = jnp.concatenate(
        [dst, jnp.full((padn,), n, jnp.int32)]).reshape(NW * cpw, CHUNK)
    packed2 = jnp.bitwise_or(jnp.left_shift(dst2, 16), src2)
    xp = jnp.pad(x, ((0, n2 - n), (0, 0)))

    degp = _make_deg(n2, cpw)(dst2)
    d0 = degp[0, :, 0:1]
    d1 = degp[1, :, 0:1]
    x1, dis, dinv = _prep(xp, d0, d1)

    hop = _make_hop(n2, cpw)
    p = hop(x1, packed2)
    x3 = _mid(p[0], p[1], x1, dinv)
    q = hop(x3, packed2)
    out = _final(q[0][:n], q[1][:n], x3[:n], dis[:n], W,
                 b.reshape(1, D), n)
    return out
